# baseline (device time: 7668 ns/iter reference)
import jax
import jax.numpy as jnp
from jax import lax
from jax.experimental import pallas as pl
from jax.experimental.pallas import tpu as pltpu

N_HALVES = 2
N_BLOCKS = 4


def kernel(x):
    m, n = x.shape
    bm = m // N_BLOCKS
    n2 = n // N_HALVES

    def body(x_ref, out_ref, acc_ref, recv_ref, send_sems, recv_sems):
        j = pl.program_id(0)
        i = pl.program_id(1)
        my_x = lax.axis_index("x")
        my_y = lax.axis_index("y")
        peer = (1 - my_x, my_y)

        @pl.when((j == 0) & (i == 0))
        def _():
            barrier_sem = pltpu.get_barrier_semaphore()
            pl.semaphore_signal(
                barrier_sem, inc=1, device_id=peer,
                device_id_type=pl.DeviceIdType.MESH,
            )
            pl.semaphore_wait(barrier_sem, 1)

        @pl.when(i == 0)
        def _():
            acc_ref[j, :] = jnp.zeros((n2,), acc_ref.dtype)

        acc_ref[j, :] += jnp.sum(x_ref[:, :], axis=0)

        @pl.when(i == N_BLOCKS - 1)
        def _():
            rdma = pltpu.make_async_remote_copy(
                src_ref=acc_ref.at[j],
                dst_ref=recv_ref.at[j],
                send_sem=send_sems.at[j],
                recv_sem=recv_sems.at[j],
                device_id=peer,
                device_id_type=pl.DeviceIdType.MESH,
            )
            rdma.start()

        @pl.when((j == N_HALVES - 1) & (i == N_BLOCKS - 1))
        def _():
            for jj in range(N_HALVES):
                wait = pltpu.make_async_remote_copy(
                    src_ref=acc_ref.at[jj],
                    dst_ref=recv_ref.at[jj],
                    send_sem=send_sems.at[jj],
                    recv_sem=recv_sems.at[jj],
                    device_id=peer,
                    device_id_type=pl.DeviceIdType.MESH,
                )
                wait.wait()
                out_ref[0, pl.ds(jj * n2, n2)] = (
                    acc_ref[jj, :] + recv_ref[jj, :]
                )

    return pl.pallas_call(
        body,
        grid=(N_HALVES, N_BLOCKS),
        out_shape=jax.ShapeDtypeStruct((1, n), x.dtype),
        in_specs=[pl.BlockSpec((bm, n2), lambda j, i: (i, j))],
        out_specs=pl.BlockSpec((1, n), lambda j, i: (0, 0)),
        scratch_shapes=[
            pltpu.VMEM((N_HALVES, n2), x.dtype),
            pltpu.VMEM((N_HALVES, n2), x.dtype),
            pltpu.SemaphoreType.DMA((N_HALVES,)),
            pltpu.SemaphoreType.DMA((N_HALVES,)),
        ],
        compiler_params=pltpu.CompilerParams(collective_id=0),
    )(x)


# device time: 7327 ns/iter; 1.0465x vs baseline; 1.0465x over previous
import jax
import jax.numpy as jnp
from jax import lax
from jax.experimental import pallas as pl
from jax.experimental.pallas import tpu as pltpu

N_HALVES = 2


def kernel(x):
    m, n = x.shape
    n2 = n // N_HALVES

    def body(x_ref, out_ref, acc_ref, recv_ref, send_sems, recv_sems):
        j = pl.program_id(0)
        my_x = lax.axis_index("x")
        my_y = lax.axis_index("y")
        peer = (1 - my_x, my_y)

        @pl.when(j == 0)
        def _():
            barrier_sem = pltpu.get_barrier_semaphore()
            pl.semaphore_signal(
                barrier_sem, inc=1, device_id=peer,
                device_id_type=pl.DeviceIdType.MESH,
            )
            pl.semaphore_wait(barrier_sem, 1)

        acc_ref[j, :] = jnp.sum(x_ref[:, :], axis=0)

        rdma = pltpu.make_async_remote_copy(
            src_ref=acc_ref.at[j],
            dst_ref=recv_ref.at[j],
            send_sem=send_sems.at[j],
            recv_sem=recv_sems.at[j],
            device_id=peer,
            device_id_type=pl.DeviceIdType.MESH,
        )
        rdma.start()

        @pl.when(j == N_HALVES - 1)
        def _():
            for jj in range(N_HALVES):
                wait = pltpu.make_async_remote_copy(
                    src_ref=acc_ref.at[jj],
                    dst_ref=recv_ref.at[jj],
                    send_sem=send_sems.at[jj],
                    recv_sem=recv_sems.at[jj],
                    device_id=peer,
                    device_id_type=pl.DeviceIdType.MESH,
                )
                wait.wait()
                out_ref[0, pl.ds(jj * n2, n2)] = (
                    acc_ref[jj, :] + recv_ref[jj, :]
                )

    return pl.pallas_call(
        body,
        grid=(N_HALVES,),
        out_shape=jax.ShapeDtypeStruct((1, n), x.dtype),
        in_specs=[pl.BlockSpec((m, n2), lambda j: (0, j))],
        out_specs=pl.BlockSpec((1, n), lambda j: (0, 0)),
        scratch_shapes=[
            pltpu.VMEM((N_HALVES, n2), x.dtype),
            pltpu.VMEM((N_HALVES, n2), x.dtype),
            pltpu.SemaphoreType.DMA((N_HALVES,)),
            pltpu.SemaphoreType.DMA((N_HALVES,)),
        ],
        compiler_params=pltpu.CompilerParams(collective_id=0),
    )(x)


# device time: 6639 ns/iter; 1.1550x vs baseline; 1.1036x over previous
import jax
import jax.numpy as jnp
from jax import lax
from jax.experimental import pallas as pl
from jax.experimental.pallas import tpu as pltpu

N_CHUNKS = 4


def kernel(x):
    m, n = x.shape
    bm = m // N_CHUNKS

    def body(x_hbm, out_ref, xv, acc, recv, copy_sems, send_sem, recv_sem):
        my_x = lax.axis_index("x")
        my_y = lax.axis_index("y")
        peer = (1 - my_x, my_y)

        def chunk_copy(k):
            return pltpu.make_async_copy(
                x_hbm.at[pl.ds(k * bm, bm), :],
                xv.at[k],
                copy_sems.at[k],
            )

        for k in range(N_CHUNKS):
            chunk_copy(k).start()

        barrier_sem = pltpu.get_barrier_semaphore()
        pl.semaphore_signal(
            barrier_sem, inc=1, device_id=peer,
            device_id_type=pl.DeviceIdType.MESH,
        )
        pl.semaphore_wait(barrier_sem, 1)

        for k in range(N_CHUNKS):
            chunk_copy(k).wait()
            if k == 0:
                acc[:, :] = jnp.sum(xv[k], axis=0, keepdims=True)
            else:
                acc[:, :] += jnp.sum(xv[k], axis=0, keepdims=True)

        rdma = pltpu.make_async_remote_copy(
            src_ref=acc,
            dst_ref=recv,
            send_sem=send_sem,
            recv_sem=recv_sem,
            device_id=peer,
            device_id_type=pl.DeviceIdType.MESH,
        )
        rdma.start()
        rdma.wait()

        out_ref[:, :] = acc[:, :] + recv[:, :]

    return pl.pallas_call(
        body,
        out_shape=jax.ShapeDtypeStruct((1, n), x.dtype),
        in_specs=[pl.BlockSpec(memory_space=pl.ANY)],
        out_specs=pl.BlockSpec(memory_space=pltpu.VMEM),
        scratch_shapes=[
            pltpu.VMEM((N_CHUNKS, bm, n), x.dtype),
            pltpu.VMEM((1, n), x.dtype),
            pltpu.VMEM((1, n), x.dtype),
            pltpu.SemaphoreType.DMA((N_CHUNKS,)),
            pltpu.SemaphoreType.DMA,
            pltpu.SemaphoreType.DMA,
        ],
        compiler_params=pltpu.CompilerParams(collective_id=0),
    )(pltpu.with_memory_space_constraint(x, pltpu.MemorySpace.HBM))
